# baseline (speedup 1.0000x reference)
import functools
import math

import jax
import jax.numpy as jnp
from jax import lax
from jax.experimental import pallas as pl
from jax.experimental.pallas import tpu as pltpu
from jax.experimental.pallas import tpu_sc as plsc

EMB = 64
LANES = 16
SCALE = math.sqrt(EMB)

NC = 2
NS = 16
NW = NC * NS

CHUNK = 128
NBUF = 4
UNROLL = 8


@functools.lru_cache(maxsize=None)
def _build(nch):
    rows_per_w = nch * CHUNK
    total = NW * rows_per_w
    mesh = plsc.VectorSubcoreMesh(core_axis_name="c", subcore_axis_name="s")

    @functools.partial(
        pl.kernel,
        mesh=mesh,
        out_type=jax.ShapeDtypeStruct((total, EMB), jnp.float32),
        scratch_types=(
            [pltpu.VMEM((nch, CHUNK), jnp.int32)]
            + [pltpu.VMEM((CHUNK, EMB), jnp.float32) for _ in range(2 * NBUF)]
            + [pltpu.SemaphoreType.DMA for _ in range(2 * NBUF)]
        ),
        compiler_params=pltpu.CompilerParams(use_tc_tiling_on_sc=False),
    )
    def emb(mask_hbm, table_hbm, out_hbm, idx_v, *rest):
        ins = rest[0:NBUF]
        outs = rest[NBUF:2 * NBUF]
        gsems = rest[2 * NBUF:3 * NBUF]
        osems = rest[3 * NBUF:4 * NBUF]

        cid = lax.axis_index("c")
        sid = lax.axis_index("s")
        wid = sid * NC + cid
        row0 = wid * rows_per_w

        pltpu.sync_copy(mask_hbm.at[wid], idx_v)

        def fire_gather(g, b):
            pltpu.async_copy(table_hbm.at[idx_v.at[g]], ins[b], gsems[b])

        def wait_gather(g, b):
            pltpu.make_async_copy(table_hbm.at[idx_v.at[g]], ins[b], gsems[b]).wait()

        def fire_out(g, b):
            pltpu.async_copy(
                outs[b], out_hbm.at[pl.ds(row0 + g * CHUNK, CHUNK)], osems[b])

        def wait_out(g, b):
            pltpu.make_async_copy(
                outs[b], out_hbm.at[pl.ds(row0 + g * CHUNK, CHUNK)], osems[b]).wait()

        def scale(b):
            def rbody(i, _):
                for dr in range(UNROLL):
                    r = i * UNROLL + dr
                    for j in range(EMB // LANES):
                        sl = pl.ds(j * LANES, LANES)
                        outs[b][r, sl] = ins[b][r, sl] * SCALE
                return 0
            lax.fori_loop(0, CHUNK // UNROLL, rbody, 0)

        ngrp = nch // NBUF

        for b in range(NBUF):
            fire_gather(b, b)
        for b in range(NBUF):
            wait_gather(b, b)
            scale(b)
            fire_out(b, b)
            fire_gather(b + NBUF, b)

        def group(gg, _):
            for b in range(NBUF):
                g = gg * NBUF + b
                wait_gather(g, b)
                wait_out(g - NBUF, b)
                scale(b)
                fire_out(g, b)
                fire_gather(g + NBUF, b)
            return 0
        lax.fori_loop(1, ngrp - 1, group, 0)

        for b in range(NBUF):
            g = (ngrp - 1) * NBUF + b
            wait_gather(g, b)
            wait_out(g - NBUF, b)
            scale(b)
            fire_out(g, b)
        for b in range(NBUF):
            g = (ngrp - 1) * NBUF + b
            wait_out(g, b)

    return emb


def kernel(mask, weight):
    bsz, hist = mask.shape
    total = bsz * hist
    idx = mask.reshape(total).astype(jnp.int32)
    rows_per_w = total // NW
    nch = rows_per_w // CHUNK
    idx3 = idx.reshape(NW, nch, CHUNK)
    out = _build(nch)(idx3, weight)
    return out.reshape(bsz, hist, EMB)
